# Initial kernel scaffold; baseline (speedup 1.0000x reference)
import jax
import jax.numpy as jnp
from jax.experimental import pallas as pl


def kernel(feature, W):
    raise NotImplementedError("write your pallas kernel here")


# speedup vs baseline: 2.9098x; 2.9098x over previous
import jax
import jax.numpy as jnp
from jax.experimental import pallas as pl
from jax.experimental.pallas import tpu as pltpu

EMB = 32
GROUPS = 8
CODES = 1024
BLOCK_B = 1024


def _vq_body(f_ref, wt_ref, w2_ref, w_ref, out_ref, loss_ref):
    wt = wt_ref[...]
    w2 = w2_ref[...]
    w = w_ref[...]
    loss_acc = jnp.zeros((f_ref.shape[0], 1), jnp.float32)
    for g in range(GROUPS):
        f = f_ref[:, g * EMB:(g + 1) * EMB]
        s = jax.lax.dot_general(
            f, wt, (((1,), (0,)), ((), ())),
            preferred_element_type=jnp.float32)
        q = w2 - 2.0 * s
        m = jnp.min(q, axis=1, keepdims=True)
        iota = jax.lax.broadcasted_iota(jnp.int32, q.shape, 1)
        j = jnp.min(jnp.where(q == m, iota, CODES), axis=1, keepdims=True)
        onehot = (iota == j).astype(jnp.float32)
        out_ref[:, g * EMB:(g + 1) * EMB] = jax.lax.dot_general(
            onehot, w, (((1,), (0,)), ((), ())),
            preferred_element_type=jnp.float32)
        f2 = jnp.sum(f * f, axis=1, keepdims=True)
        loss_acc = loss_acc + (f2 + m)
    loss_ref[...] = (1.25 / GROUPS) * loss_acc


def kernel(feature, W):
    n, fl = feature.shape
    wt = W.T
    w2 = jnp.sum(W * W, axis=1)[None, :]
    grid = n // BLOCK_B
    out, loss = pl.pallas_call(
        _vq_body,
        grid=(grid,),
        in_specs=[
            pl.BlockSpec((BLOCK_B, fl), lambda i: (i, 0)),
            pl.BlockSpec((EMB, CODES), lambda i: (0, 0)),
            pl.BlockSpec((1, CODES), lambda i: (0, 0)),
            pl.BlockSpec((CODES, EMB), lambda i: (0, 0)),
        ],
        out_specs=[
            pl.BlockSpec((BLOCK_B, fl), lambda i: (i, 0)),
            pl.BlockSpec((BLOCK_B, 1), lambda i: (i, 0)),
        ],
        out_shape=[
            jax.ShapeDtypeStruct((n, fl), jnp.float32),
            jax.ShapeDtypeStruct((n, 1), jnp.float32),
        ],
    )(feature, wt, w2, W)
    return (loss[:, 0], out)


# speedup vs baseline: 3.0382x; 1.0441x over previous
import functools
import jax
import jax.numpy as jnp
from jax import lax
from jax.experimental import pallas as pl
from jax.experimental.pallas import tpu as pltpu
from jax.experimental.pallas import tpu_sc as plsc

EMB = 32
GROUPS = 8
CODES = 1024
BLOCK_B = 1024
IDX_MINOR = 128


def _vq_argmin_body(f_ref, wt_ref, w2_ref, j_ref, loss_ref):
    wt = wt_ref[...]
    w2 = w2_ref[...]
    loss_acc = jnp.zeros((f_ref.shape[0], 1), jnp.float32)
    js = []
    for g in range(GROUPS):
        f = f_ref[:, g * EMB:(g + 1) * EMB]
        s = jax.lax.dot_general(
            f, wt, (((1,), (0,)), ((), ())),
            preferred_element_type=jnp.float32)
        q = w2 - 2.0 * s
        m = jnp.min(q, axis=1, keepdims=True)
        iota = jax.lax.broadcasted_iota(jnp.int32, q.shape, 1)
        j = jnp.min(jnp.where(q == m, iota, CODES), axis=1, keepdims=True)
        js.append(j)
        f2 = jnp.sum(f * f, axis=1, keepdims=True)
        loss_acc = loss_acc + (f2 + m)
    j_ref[...] = jnp.concatenate(js, axis=1)
    loss_ref[...] = (1.25 / GROUPS) * loss_acc


def _argmin_call(feature, wt, w2):
    n, fl = feature.shape
    grid = n // BLOCK_B
    return pl.pallas_call(
        _vq_argmin_body,
        grid=(grid,),
        in_specs=[
            pl.BlockSpec((BLOCK_B, fl), lambda i: (i, 0)),
            pl.BlockSpec((EMB, CODES), lambda i: (0, 0)),
            pl.BlockSpec((1, CODES), lambda i: (0, 0)),
        ],
        out_specs=[
            pl.BlockSpec((BLOCK_B, GROUPS), lambda i: (i, 0)),
            pl.BlockSpec((BLOCK_B, 1), lambda i: (i, 0)),
        ],
        out_shape=[
            jax.ShapeDtypeStruct((n, GROUPS), jnp.int32),
            jax.ShapeDtypeStruct((n, 1), jnp.float32),
        ],
    )(feature, wt, w2)


def _make_sc_gather(nrows):
    info = plsc.get_sparse_core_info()
    nw = info.num_cores * info.num_subcores
    b_per_w = nrows // nw
    ir_per_w = b_per_w // IDX_MINOR
    half = b_per_w // 2
    mesh = plsc.VectorSubcoreMesh(core_axis_name="c", subcore_axis_name="s")

    @functools.partial(
        pl.kernel, mesh=mesh,
        out_type=jax.ShapeDtypeStruct((nrows, EMB), jnp.float32),
        scratch_types=[
            pltpu.VMEM((ir_per_w, IDX_MINOR), jnp.int32),
            pltpu.VMEM((half, EMB), jnp.float32),
            pltpu.SemaphoreType.DMA,
        ],
        compiler_params=pltpu.CompilerParams(use_tc_tiling_on_sc=False),
    )
    def gather_k(idx_hbm, w_hbm, out_hbm, idx_v, rows_v, sem):
        wid = lax.axis_index("s") * info.num_cores + lax.axis_index("c")
        base = wid * b_per_w
        pltpu.sync_copy(idx_hbm.at[pl.ds(wid * ir_per_w, ir_per_w)], idx_v)
        for h in range(2):
            n_streams = half // IDX_MINOR
            copies = []
            for c in range(n_streams):
                copies.append(pltpu.async_copy(
                    w_hbm.at[idx_v.at[h * n_streams + c]],
                    rows_v.at[pl.ds(c * IDX_MINOR, IDX_MINOR)],
                    sem))
            for cp in copies:
                cp.wait()
            pltpu.sync_copy(rows_v, out_hbm.at[pl.ds(base + h * half, half)])

    return gather_k


def kernel(feature, W):
    n, fl = feature.shape
    wt = W.T
    w2 = jnp.sum(W * W, axis=1)[None, :]
    j, loss = _argmin_call(feature, wt, w2)
    nrows = n * GROUPS
    gather_k = _make_sc_gather(nrows)
    rows = gather_k(j.reshape(nrows // IDX_MINOR, IDX_MINOR), W)
    return (loss[:, 0], rows.reshape(n, fl))


# speedup vs baseline: 3.1979x; 1.0526x over previous
import functools
import jax
import jax.numpy as jnp
from jax import lax
from jax.experimental import pallas as pl
from jax.experimental.pallas import tpu as pltpu
from jax.experimental.pallas import tpu_sc as plsc

EMB = 32
GROUPS = 8
CODES = 1024
BLOCK_B = 1024
IDX_MINOR = 128
SLICES = 2


def _vq_argmin_body(f_ref, wtn_ref, w2_ref, iota_ref, j_ref, loss_ref):
    wtn = wtn_ref[...]
    w2 = w2_ref[...]
    b = f_ref.shape[0]
    iota_f = iota_ref[...]
    loss_acc = jnp.zeros((b, 1), jnp.float32)
    js = []
    for g in range(GROUPS):
        f = f_ref[:, g * EMB:(g + 1) * EMB]
        s = jax.lax.dot_general(
            f, wtn, (((1,), (0,)), ((), ())),
            preferred_element_type=jnp.float32)
        q = s + w2
        m = jnp.min(q, axis=1, keepdims=True)
        jf = jnp.min(jnp.where(q == m, iota_f, float(CODES)),
                     axis=1, keepdims=True)
        js.append(jf)
        f2 = jnp.sum(f * f, axis=1, keepdims=True)
        loss_acc = loss_acc + (f2 + m)
    j_ref[...] = jnp.concatenate(js, axis=1).astype(jnp.int32)
    loss_ref[...] = jnp.reshape((1.25 / GROUPS) * loss_acc, (b,))


def _argmin_call(feature, wtn, w2, iota_f, n_half, block_off):
    fl = feature.shape[1]
    grid = n_half // BLOCK_B
    return pl.pallas_call(
        _vq_argmin_body,
        grid=(grid,),
        in_specs=[
            pl.BlockSpec((BLOCK_B, fl), lambda i: (i + block_off, 0)),
            pl.BlockSpec((EMB, CODES), lambda i: (0, 0)),
            pl.BlockSpec((1, CODES), lambda i: (0, 0)),
            pl.BlockSpec((1, CODES), lambda i: (0, 0)),
        ],
        out_specs=[
            pl.BlockSpec((BLOCK_B, GROUPS), lambda i: (i, 0)),
            pl.BlockSpec((BLOCK_B,), lambda i: (i,)),
        ],
        out_shape=[
            jax.ShapeDtypeStruct((n_half, GROUPS), jnp.int32),
            jax.ShapeDtypeStruct((n_half,), jnp.float32),
        ],
    )(feature, wtn, w2, iota_f)


def _make_sc_gather(nrows_half, out_base):
    info = plsc.get_sparse_core_info()
    nw = info.num_cores * info.num_subcores
    b_per_w = nrows_half // nw
    ir_per_w = b_per_w // IDX_MINOR
    n_chunks = 4
    chunk = b_per_w // n_chunks
    spc = chunk // IDX_MINOR
    mesh = plsc.VectorSubcoreMesh(core_axis_name="c", subcore_axis_name="s")

    @functools.partial(
        pl.kernel, mesh=mesh,
        out_type=(),
        scratch_types=[
            pltpu.VMEM((ir_per_w, IDX_MINOR), jnp.int32),
            pltpu.VMEM((chunk, EMB), jnp.float32),
            pltpu.VMEM((chunk, EMB), jnp.float32),
            pltpu.SemaphoreType.DMA,
            pltpu.SemaphoreType.DMA,
        ],
        compiler_params=pltpu.CompilerParams(use_tc_tiling_on_sc=False),
    )
    def gather_k(idx_hbm, w_hbm, out_hbm, idx_v, rows_a, rows_b, sg, sw):
        wid = lax.axis_index("s") * info.num_cores + lax.axis_index("c")
        base = out_base + wid * b_per_w
        bufs = (rows_a, rows_b)
        pltpu.sync_copy(idx_hbm.at[pl.ds(wid * ir_per_w, ir_per_w)], idx_v)

        def fire(c, buf):
            for t in range(spc):
                pltpu.async_copy(
                    w_hbm.at[idx_v.at[c * spc + t]],
                    buf.at[pl.ds(t * IDX_MINOR, IDX_MINOR)],
                    sg)

        wb = []
        fire(0, bufs[0])
        for c in range(n_chunks):
            buf = bufs[c % 2]
            for t in range(spc):
                pltpu.make_async_copy(
                    w_hbm.at[idx_v.at[c * spc + t]],
                    buf.at[pl.ds(t * IDX_MINOR, IDX_MINOR)],
                    sg).wait()
            if c + 1 < n_chunks:
                if c >= 1:
                    wb.pop(0).wait()
                fire(c + 1, bufs[(c + 1) % 2])
            wb.append(pltpu.async_copy(
                buf, out_hbm.at[pl.ds(base + c * chunk, chunk)], sw))
        for h in wb:
            h.wait()

    return gather_k


def kernel(feature, W):
    n, fl = feature.shape
    wtn = -2.0 * W.T
    w2 = jnp.sum(W * W, axis=1)[None, :]
    iota_f = jnp.arange(CODES, dtype=jnp.float32)[None, :]
    n_half = n // SLICES
    nrows_half = n_half * GROUPS
    out_ref = jax.new_ref(jnp.zeros((n * GROUPS, EMB), jnp.float32))
    losses = []
    for s in range(SLICES):
        gather_k = _make_sc_gather(nrows_half, s * nrows_half)
        j, loss = _argmin_call(
            feature, wtn, w2, iota_f, n_half, s * (n_half // BLOCK_B))
        gather_k(j.reshape(nrows_half // IDX_MINOR, IDX_MINOR), W, out_ref)
        losses.append(loss)
    return (jnp.concatenate(losses), out_ref[...].reshape(n, fl))
